# baseline (speedup 1.0000x reference)
import functools

import jax
import jax.numpy as jnp
from jax import lax
from jax.experimental import pallas as pl
from jax.experimental.pallas import tpu as pltpu
from jax.experimental.pallas import tpu_sc as plsc

_B, _S, _D, _V = 1024, 50, 512, 100000
_NC, _NS = 2, 16
_NW = _NC * _NS
_BPW = _B // _NW
_NCHUNK = _BPW
_CHUNK = _S
_LANES = 16
_H = 8
_T = _D // 128
_SP = 56
_CR = _T * _SP
_IPW = _BPW * _S
_IDXPAD = 1664


def _pe_table():
    i = jnp.arange(_S, dtype=jnp.float32)[:, None]
    j = jnp.arange(_D // 2, dtype=jnp.float32)[None, :]
    ang = i / jnp.power(jnp.float32(10000.0), 2.0 * j / _D)
    pe = jnp.zeros((_S, _D), dtype=jnp.float32)
    pe = pe.at[:, 0::2].set(jnp.sin(ang))
    pe = pe.at[:, 1::2].set(jnp.cos(ang))
    pe_t = pe.reshape(_S, _T, 128).transpose(1, 0, 2)
    pe_t = jnp.pad(pe_t, ((0, 0), (0, _SP - _S), (0, 0)))
    return pe_t.reshape(_CR, 128)


_mesh = plsc.VectorSubcoreMesh(core_axis_name="c", subcore_axis_name="s")


@functools.partial(
    pl.kernel,
    mesh=_mesh,
    out_type=jax.ShapeDtypeStruct((_B, _CR, 128), jnp.float32),
    scratch_types=[
        pltpu.VMEM((_IDXPAD,), jnp.int32),
        pltpu.VMEM((_NCHUNK, _T, 64), jnp.int32),
        pltpu.VMEM((_CR, 128), jnp.float32),
        pltpu.VMEM((2, _CR, 128), jnp.float32),
        pltpu.SemaphoreType.DMA,
        pltpu.SemaphoreType.DMA,
    ],
    compiler_params=pltpu.CompilerParams(use_tc_tiling_on_sc=False),
)
def _sc_lookup(x_hbm, pe_hbm, tbl_hbm, out_hbm, idx_v, seg_v, pe_v, rows_v,
               gsem, ssem):
    wid = lax.axis_index("s") * _NC + lax.axis_index("c")
    pltpu.sync_copy(x_hbm.at[wid], idx_v.at[pl.ds(0, _IPW)])
    pltpu.sync_copy(pe_hbm, pe_v)

    def expand(j, _):
        for m in range(4):
            pos = lax.iota(jnp.int32, _LANES) + (m * _LANES)
            live = pos < _CHUNK
            g = idx_v[pl.ds(j * _CHUNK + m * _LANES, _LANES)]
            base = jnp.where(live, ((g >> 3) << 5) + (g & 7), 0)
            for t in range(_T):
                seg_v[j, t, pl.ds(m * _LANES, _LANES)] = jnp.where(
                    live, base + t * _H, 0)
        return _

    lax.fori_loop(0, _NCHUNK, expand, 0)

    def add_pe(i, buf):
        for t in range(_T):
            for r in range(2):
                row = t * _SP + 2 * i + r
                for q in range(8):
                    sl = pl.ds(q * _LANES, _LANES)
                    plsc.addupdate(rows_v.at[buf, row, sl], pe_v[row, sl])
        return buf

    def chunk_body(j, _):
        b = lax.rem(j, 2)
        for t in range(_T):
            pltpu.async_copy(
                tbl_hbm.at[seg_v.at[j, t, pl.ds(0, _SP)]],
                rows_v.at[b, pl.ds(t * _SP, _SP)],
                gsem,
            ).wait()
        lax.fori_loop(0, _CHUNK // 2, add_pe, b)
        pltpu.async_copy(rows_v.at[b], out_hbm.at[wid * _NCHUNK + j], ssem).wait()
        return _

    lax.fori_loop(0, _NCHUNK, chunk_body, 0)


def kernel(x, offsets, table):
    del offsets
    x3 = x.reshape(_NW, _IPW)
    t2 = (table.reshape(_V // _H, _H, _T, 128)
          .transpose(0, 2, 1, 3)
          .reshape(_V * _T, 128))
    out = _sc_lookup(x3, _pe_table(), t2)
    return (out.reshape(_B, _T, _SP, 128)[:, :, :_S, :]
            .transpose(0, 2, 1, 3)
            .reshape(_B, _S, _D))
